# baseline (speedup 1.0000x reference)
import functools

import jax
import jax.numpy as jnp
from jax import lax
from jax.experimental import pallas as pl
from jax.experimental.pallas import tpu as pltpu
from jax.experimental.pallas import tpu_sc as plsc

N = 10000
E = 320000
D = 128
K = 3

NC = 2
NS = 16
CHUNK = 128
EP = 327680
EROWS = EP // CHUNK
ROWS_PER_TILE = EROWS // (NC * NS)
NPAD = 10112
ZR = NPAD // NS
OR_HI = 632
OR_LAST = N - 15 * OR_HI
DUMMY = N


HALF = ROWS_PER_TILE // 2


@functools.cache
def _sc_calls():
    mesh = plsc.VectorSubcoreMesh(core_axis_name="c", subcore_axis_name="s",
                                  num_cores=NC, num_subcores=NS)
    segsum_call = pl.kernel(
        _segsum_body,
        out_type=jax.ShapeDtypeStruct((NC, N, D), jnp.float32),
        mesh=mesh,
        scratch_types=[
            pltpu.VMEM((HALF, CHUNK), jnp.int32),
            pltpu.VMEM((HALF, CHUNK), jnp.int32),
            pltpu.VMEM((CHUNK, D), jnp.float32),
            pltpu.VMEM((CHUNK, D), jnp.float32),
            pltpu.VMEM_SHARED((NPAD, D), jnp.float32),
            pltpu.SemaphoreType.DMA,
            pltpu.SemaphoreType.DMA,
        ],
    )
    deg_call = pl.kernel(
        _deg_body,
        out_type=jax.ShapeDtypeStruct((NC, N, D), jnp.float32),
        mesh=mesh,
        scratch_types=[
            pltpu.VMEM((ROWS_PER_TILE, CHUNK), jnp.int32),
            pltpu.VMEM((CHUNK, D), jnp.float32),
            pltpu.VMEM_SHARED((NPAD, D), jnp.float32),
            pltpu.SemaphoreType.DMA,
        ],
    )
    return deg_call, segsum_call


def _segsum_body(p_hbm, src_hbm, dst_hbm, zeros_hbm, out_hbm,
                 src_v, dst_v, rows_a, rows_b, acc, sem_a, sem_b):
    c = lax.axis_index("c")
    s = lax.axis_index("s")
    w = c * NS + s
    pltpu.sync_copy(zeros_hbm.at[pl.ds(s * ZR, ZR)], acc.at[pl.ds(s * ZR, ZR)])
    plsc.subcore_barrier()

    for half in range(2):
        base = w * ROWS_PER_TILE + half * HALF
        pltpu.sync_copy(src_hbm.at[pl.ds(base, HALF)], src_v)
        pltpu.sync_copy(dst_hbm.at[pl.ds(base, HALF)], dst_v)
        pltpu.async_copy(p_hbm.at[src_v.at[0]], rows_a, sem_a)

        def body(j, carry):
            even = lax.rem(j, 2) == 0
            more = j + 1 < HALF

            @pl.when(jnp.logical_and(even, more))
            def _():
                pltpu.async_copy(p_hbm.at[src_v.at[j + 1]], rows_b, sem_b)

            @pl.when(jnp.logical_and(jnp.logical_not(even), more))
            def _():
                pltpu.async_copy(p_hbm.at[src_v.at[j + 1]], rows_a, sem_a)

            @pl.when(even)
            def _():
                pltpu.make_async_copy(p_hbm.at[src_v.at[j]], rows_a, sem_a).wait()
                pltpu.sync_copy(rows_a, acc.at[dst_v.at[j]], add=True)

            @pl.when(jnp.logical_not(even))
            def _():
                pltpu.make_async_copy(p_hbm.at[src_v.at[j]], rows_b, sem_b).wait()
                pltpu.sync_copy(rows_b, acc.at[dst_v.at[j]], add=True)

            return carry

        lax.fori_loop(0, HALF, body, 0)
    plsc.subcore_barrier()

    @pl.when(s < NS - 1)
    def _():
        pltpu.sync_copy(acc.at[pl.ds(s * OR_HI, OR_HI)],
                        out_hbm.at[c, pl.ds(s * OR_HI, OR_HI)])

    @pl.when(s == NS - 1)
    def _():
        pltpu.sync_copy(acc.at[pl.ds((NS - 1) * OR_HI, OR_LAST)],
                        out_hbm.at[c, pl.ds((NS - 1) * OR_HI, OR_LAST)])


def _deg_body(ones_hbm, dst_hbm, zeros_hbm, out_hbm, dst_v, ones_v, acc, sem):
    c = lax.axis_index("c")
    s = lax.axis_index("s")
    w = c * NS + s
    pltpu.sync_copy(zeros_hbm.at[pl.ds(s * ZR, ZR)], acc.at[pl.ds(s * ZR, ZR)])
    pltpu.sync_copy(ones_hbm, ones_v)
    pltpu.sync_copy(dst_hbm.at[pl.ds(w * ROWS_PER_TILE, ROWS_PER_TILE)], dst_v)
    plsc.subcore_barrier()

    def body(j, carry):
        pltpu.sync_copy(ones_v, acc.at[dst_v.at[j]], add=True)
        return carry

    lax.fori_loop(0, ROWS_PER_TILE, body, 0)
    plsc.subcore_barrier()

    @pl.when(s < NS - 1)
    def _():
        pltpu.sync_copy(acc.at[pl.ds(s * OR_HI, OR_HI)],
                        out_hbm.at[c, pl.ds(s * OR_HI, OR_HI)])

    @pl.when(s == NS - 1)
    def _():
        pltpu.sync_copy(acc.at[pl.ds((NS - 1) * OR_HI, OR_LAST)],
                        out_hbm.at[c, pl.ds((NS - 1) * OR_HI, OR_LAST)])


R = 400
G = N // R


def _prep0_body(deg_ref, x_ref, w_ref, acc_ref, p_ref, dis_ref):
    deg = deg_ref[0, :, 0] + deg_ref[1, :, 0]
    dis = jnp.where(deg > 0, lax.rsqrt(jnp.maximum(deg, 1.0)), 0.0)[:, None]
    x = x_ref[...]
    acc_ref[...] = jnp.dot(x, w_ref[...], preferred_element_type=jnp.float32)
    p_ref[...] = x * dis
    dis_ref[...] = dis


def _mid_body(t_ref, w_ref, dis_ref, accin_ref, accout_ref, pout_ref):
    dis = dis_ref[...]
    cur = (t_ref[0] + t_ref[1]) * dis
    accout_ref[...] = accin_ref[...] + jnp.dot(
        cur, w_ref[...], preferred_element_type=jnp.float32)
    pout_ref[...] = cur * dis


def _lastprep_body(t_ref, w_ref, b_ref, dis_ref, accin_ref, wn_ref,
                   accout_ref, pout_ref):
    dis = dis_ref[...]
    cur = (t_ref[0] + t_ref[1]) * dis
    h = accin_ref[...] + jnp.dot(cur, w_ref[...],
                                 preferred_element_type=jnp.float32) + b_ref[...]
    h = jnp.maximum(h, 0.0)
    accout_ref[...] = jnp.dot(h, wn_ref[...], preferred_element_type=jnp.float32)
    pout_ref[...] = h * dis


def _final_body(t_ref, w_ref, b_ref, dis_ref, accin_ref, fcw_ref, fcb_ref,
                out_ref):
    dis = dis_ref[...]
    cur = (t_ref[0] + t_ref[1]) * dis
    h = accin_ref[...] + jnp.dot(cur, w_ref[...],
                                 preferred_element_type=jnp.float32) + b_ref[...]
    out_ref[...] = jnp.dot(h, fcw_ref[...],
                           preferred_element_type=jnp.float32) + fcb_ref[...]


def _rows(i):
    return (i, 0)


_b_t = pl.BlockSpec((NC, R, D), lambda i: (0, i, 0))
_b_x = pl.BlockSpec((R, D), _rows)
_b_w = pl.BlockSpec((D, D), lambda i: (0, 0))
_b_dis = pl.BlockSpec((R, 1), _rows)
_b_b = pl.BlockSpec((1, D), lambda i: (0, 0))

_f32 = jnp.float32
_nd = jax.ShapeDtypeStruct((N, D), _f32)

_prep0 = pl.pallas_call(
    _prep0_body, grid=(G,),
    in_specs=[_b_t, _b_x, _b_w],
    out_specs=[_b_x, _b_x, _b_dis],
    out_shape=[_nd, _nd, jax.ShapeDtypeStruct((N, 1), _f32)],
)

_mid = pl.pallas_call(
    _mid_body, grid=(G,),
    in_specs=[_b_t, _b_w, _b_dis, _b_x],
    out_specs=[_b_x, _b_x],
    out_shape=[_nd, _nd],
)

_lastprep = pl.pallas_call(
    _lastprep_body, grid=(G,),
    in_specs=[_b_t, _b_w, _b_b, _b_dis, _b_x, _b_w],
    out_specs=[_b_x, _b_x],
    out_shape=[_nd, _nd],
)

_final = pl.pallas_call(
    _final_body, grid=(G,),
    in_specs=[_b_t, _b_w, _b_b, _b_dis, _b_x, _b_w, _b_b],
    out_specs=_b_x,
    out_shape=_nd,
)


def kernel(x, edge_index, W0, b0, W1, b1, W2, b2, fc_W, fc_b):
    src = edge_index[0].astype(jnp.int32)
    dst = edge_index[1].astype(jnp.int32)
    src, order = lax.sort_key_val(src, jnp.arange(E, dtype=jnp.int32))
    dst = dst[order]
    pad = EP - E
    src2d = jnp.concatenate([src, jnp.zeros((pad,), jnp.int32)]).reshape(EROWS, CHUNK)
    dst2d = jnp.concatenate([dst, jnp.full((pad,), DUMMY, jnp.int32)]).reshape(EROWS, CHUNK)
    zerosD = jnp.zeros((NPAD, D), _f32)
    onesCD = jnp.ones((CHUNK, D), _f32)
    _deg_call, _segsum_call = _sc_calls()

    degs = _deg_call(onesCD, dst2d, zerosD)
    b0r = b0.reshape(1, D)
    b1r = b1.reshape(1, D)
    b2r = b2.reshape(1, D)
    fcbr = fc_b.reshape(1, D)

    acc, p, dis = _prep0(degs, x, W0[0])
    t = _segsum_call(p, src2d, dst2d, zerosD)
    acc, p = _mid(t, W0[1], dis, acc)
    t = _segsum_call(p, src2d, dst2d, zerosD)
    acc, p = _mid(t, W0[2], dis, acc)
    t = _segsum_call(p, src2d, dst2d, zerosD)
    acc, p = _lastprep(t, W0[3], b0r, dis, acc, W1[0])
    t = _segsum_call(p, src2d, dst2d, zerosD)
    acc, p = _mid(t, W1[1], dis, acc)
    t = _segsum_call(p, src2d, dst2d, zerosD)
    acc, p = _mid(t, W1[2], dis, acc)
    t = _segsum_call(p, src2d, dst2d, zerosD)
    acc, p = _lastprep(t, W1[3], b1r, dis, acc, W2[0])
    t = _segsum_call(p, src2d, dst2d, zerosD)
    acc, p = _mid(t, W2[1], dis, acc)
    t = _segsum_call(p, src2d, dst2d, zerosD)
    acc, p = _mid(t, W2[2], dis, acc)
    t = _segsum_call(p, src2d, dst2d, zerosD)
    out = _final(t, W2[3], b2r, dis, acc, fc_W, fcbr)
    return out


# speedup vs baseline: 1.2258x; 1.2258x over previous
import functools

import jax
import jax.numpy as jnp
from jax import lax
from jax.experimental import pallas as pl
from jax.experimental.pallas import tpu as pltpu
from jax.experimental.pallas import tpu_sc as plsc

N = 10000
E = 320000
D = 128
K = 3

NC = 2
NS = 16
CHUNK = 128
EP = 327680
EROWS = EP // CHUNK
ROWS_PER_TILE = EROWS // (NC * NS)
NPAD = 10112
ZR = NPAD // NS
OR_HI = 632
OR_LAST = N - 15 * OR_HI
DUMMY = N


HALF = ROWS_PER_TILE // 2


@functools.cache
def _sc_calls():
    mesh = plsc.VectorSubcoreMesh(core_axis_name="c", subcore_axis_name="s",
                                  num_cores=NC, num_subcores=NS)
    segsum_call = pl.kernel(
        _segsum_body,
        out_type=jax.ShapeDtypeStruct((NC, N, D), jnp.float32),
        mesh=mesh,
        scratch_types=[
            pltpu.VMEM((HALF, CHUNK), jnp.int32),
            pltpu.VMEM((HALF, CHUNK), jnp.int32),
            pltpu.VMEM((CHUNK, D), jnp.float32),
            pltpu.VMEM((CHUNK, D), jnp.float32),
            pltpu.VMEM_SHARED((NPAD, D), jnp.float32),
            pltpu.SemaphoreType.DMA,
            pltpu.SemaphoreType.DMA,
        ],
    )
    deg_call = pl.kernel(
        _deg_body,
        out_type=jax.ShapeDtypeStruct((NC, N, D), jnp.float32),
        mesh=mesh,
        scratch_types=[
            pltpu.VMEM((ROWS_PER_TILE, CHUNK), jnp.int32),
            pltpu.VMEM((CHUNK, D), jnp.float32),
            pltpu.VMEM_SHARED((NPAD, D), jnp.float32),
            pltpu.SemaphoreType.DMA,
        ],
    )
    return deg_call, segsum_call


def _segsum_body(p_hbm, src_hbm, dst_hbm, zeros_hbm, out_hbm,
                 src_v, dst_v, rows_a, rows_b, acc, sem_a, sem_b):
    c = lax.axis_index("c")
    s = lax.axis_index("s")
    w = c * NS + s
    pltpu.sync_copy(zeros_hbm.at[pl.ds(s * ZR, ZR)], acc.at[pl.ds(s * ZR, ZR)])
    plsc.subcore_barrier()

    for half in range(2):
        base = w * ROWS_PER_TILE + half * HALF
        pltpu.sync_copy(src_hbm.at[pl.ds(base, HALF)], src_v)
        pltpu.sync_copy(dst_hbm.at[pl.ds(base, HALF)], dst_v)
        pltpu.async_copy(p_hbm.at[src_v.at[0]], rows_a, sem_a)

        def body(j, carry):
            even = lax.rem(j, 2) == 0
            more = j + 1 < HALF

            @pl.when(jnp.logical_and(even, more))
            def _():
                pltpu.async_copy(p_hbm.at[src_v.at[j + 1]], rows_b, sem_b)

            @pl.when(jnp.logical_and(jnp.logical_not(even), more))
            def _():
                pltpu.async_copy(p_hbm.at[src_v.at[j + 1]], rows_a, sem_a)

            @pl.when(even)
            def _():
                pltpu.make_async_copy(p_hbm.at[src_v.at[j]], rows_a, sem_a).wait()
                pltpu.sync_copy(rows_a, acc.at[dst_v.at[j]], add=True)

            @pl.when(jnp.logical_not(even))
            def _():
                pltpu.make_async_copy(p_hbm.at[src_v.at[j]], rows_b, sem_b).wait()
                pltpu.sync_copy(rows_b, acc.at[dst_v.at[j]], add=True)

            return carry

        lax.fori_loop(0, HALF, body, 0)
    plsc.subcore_barrier()

    @pl.when(s < NS - 1)
    def _():
        pltpu.sync_copy(acc.at[pl.ds(s * OR_HI, OR_HI)],
                        out_hbm.at[c, pl.ds(s * OR_HI, OR_HI)])

    @pl.when(s == NS - 1)
    def _():
        pltpu.sync_copy(acc.at[pl.ds((NS - 1) * OR_HI, OR_LAST)],
                        out_hbm.at[c, pl.ds((NS - 1) * OR_HI, OR_LAST)])


def _deg_body(ones_hbm, dst_hbm, zeros_hbm, out_hbm, dst_v, ones_v, acc, sem):
    c = lax.axis_index("c")
    s = lax.axis_index("s")
    w = c * NS + s
    pltpu.sync_copy(zeros_hbm.at[pl.ds(s * ZR, ZR)], acc.at[pl.ds(s * ZR, ZR)])
    pltpu.sync_copy(ones_hbm, ones_v)
    pltpu.sync_copy(dst_hbm.at[pl.ds(w * ROWS_PER_TILE, ROWS_PER_TILE)], dst_v)
    plsc.subcore_barrier()

    def body(j, carry):
        pltpu.sync_copy(ones_v, acc.at[dst_v.at[j]], add=True)
        return carry

    lax.fori_loop(0, ROWS_PER_TILE, body, 0)
    plsc.subcore_barrier()

    @pl.when(s < NS - 1)
    def _():
        pltpu.sync_copy(acc.at[pl.ds(s * OR_HI, OR_HI)],
                        out_hbm.at[c, pl.ds(s * OR_HI, OR_HI)])

    @pl.when(s == NS - 1)
    def _():
        pltpu.sync_copy(acc.at[pl.ds((NS - 1) * OR_HI, OR_LAST)],
                        out_hbm.at[c, pl.ds((NS - 1) * OR_HI, OR_LAST)])


R = 400
G = N // R


def _prep0_body(deg_ref, x_ref, w_ref, acc_ref, p_ref, dis_ref):
    deg = deg_ref[0, :, 0] + deg_ref[1, :, 0]
    dis = jnp.where(deg > 0, lax.rsqrt(jnp.maximum(deg, 1.0)), 0.0)[:, None]
    x = x_ref[...]
    acc_ref[...] = jnp.dot(x, w_ref[...], preferred_element_type=jnp.float32)
    p_ref[...] = x * dis
    dis_ref[...] = dis


def _mid_body(t_ref, w_ref, dis_ref, accin_ref, accout_ref, pout_ref):
    dis = dis_ref[...]
    cur = (t_ref[0] + t_ref[1]) * dis
    accout_ref[...] = accin_ref[...] + jnp.dot(
        cur, w_ref[...], preferred_element_type=jnp.float32)
    pout_ref[...] = cur * dis


def _lastprep_body(t_ref, w_ref, b_ref, dis_ref, accin_ref, wn_ref,
                   accout_ref, pout_ref):
    dis = dis_ref[...]
    cur = (t_ref[0] + t_ref[1]) * dis
    h = accin_ref[...] + jnp.dot(cur, w_ref[...],
                                 preferred_element_type=jnp.float32) + b_ref[...]
    h = jnp.maximum(h, 0.0)
    accout_ref[...] = jnp.dot(h, wn_ref[...], preferred_element_type=jnp.float32)
    pout_ref[...] = h * dis


def _final_body(t_ref, w_ref, b_ref, dis_ref, accin_ref, fcw_ref, fcb_ref,
                out_ref):
    dis = dis_ref[...]
    cur = (t_ref[0] + t_ref[1]) * dis
    h = accin_ref[...] + jnp.dot(cur, w_ref[...],
                                 preferred_element_type=jnp.float32) + b_ref[...]
    out_ref[...] = jnp.dot(h, fcw_ref[...],
                           preferred_element_type=jnp.float32) + fcb_ref[...]


def _rows(i):
    return (i, 0)


_b_t = pl.BlockSpec((NC, R, D), lambda i: (0, i, 0))
_b_x = pl.BlockSpec((R, D), _rows)
_b_w = pl.BlockSpec((D, D), lambda i: (0, 0))
_b_dis = pl.BlockSpec((R, 1), _rows)
_b_b = pl.BlockSpec((1, D), lambda i: (0, 0))

_f32 = jnp.float32
_nd = jax.ShapeDtypeStruct((N, D), _f32)

_prep0 = pl.pallas_call(
    _prep0_body, grid=(G,),
    in_specs=[_b_t, _b_x, _b_w],
    out_specs=[_b_x, _b_x, _b_dis],
    out_shape=[_nd, _nd, jax.ShapeDtypeStruct((N, 1), _f32)],
)

_mid = pl.pallas_call(
    _mid_body, grid=(G,),
    in_specs=[_b_t, _b_w, _b_dis, _b_x],
    out_specs=[_b_x, _b_x],
    out_shape=[_nd, _nd],
)

_lastprep = pl.pallas_call(
    _lastprep_body, grid=(G,),
    in_specs=[_b_t, _b_w, _b_b, _b_dis, _b_x, _b_w],
    out_specs=[_b_x, _b_x],
    out_shape=[_nd, _nd],
)

_final = pl.pallas_call(
    _final_body, grid=(G,),
    in_specs=[_b_t, _b_w, _b_b, _b_dis, _b_x, _b_w, _b_b],
    out_specs=_b_x,
    out_shape=_nd,
)


def kernel(x, edge_index, W0, b0, W1, b1, W2, b2, fc_W, fc_b):
    src = edge_index[0].astype(jnp.int32)
    dst = edge_index[1].astype(jnp.int32)
    pad = EP - E
    src2d = jnp.concatenate([src, jnp.zeros((pad,), jnp.int32)]).reshape(EROWS, CHUNK)
    src2d = src2d + jnp.concatenate(
        [jnp.zeros((EROWS // 2, CHUNK), jnp.int32),
         jnp.full((EROWS // 2, CHUNK), N, jnp.int32)])
    dst2d = jnp.concatenate([dst, jnp.full((pad,), DUMMY, jnp.int32)]).reshape(EROWS, CHUNK)
    zerosD = jnp.zeros((NPAD, D), _f32)
    onesCD = jnp.ones((CHUNK, D), _f32)
    _deg_call, _segsum_call = _sc_calls()

    degs = _deg_call(onesCD, dst2d, zerosD)
    b0r = b0.reshape(1, D)
    b1r = b1.reshape(1, D)
    b2r = b2.reshape(1, D)
    fcbr = fc_b.reshape(1, D)

    acc, p, dis = _prep0(degs, x, W0[0])
    t = _segsum_call(jnp.concatenate([p, p]), src2d, dst2d, zerosD)
    acc, p = _mid(t, W0[1], dis, acc)
    t = _segsum_call(jnp.concatenate([p, p]), src2d, dst2d, zerosD)
    acc, p = _mid(t, W0[2], dis, acc)
    t = _segsum_call(jnp.concatenate([p, p]), src2d, dst2d, zerosD)
    acc, p = _lastprep(t, W0[3], b0r, dis, acc, W1[0])
    t = _segsum_call(jnp.concatenate([p, p]), src2d, dst2d, zerosD)
    acc, p = _mid(t, W1[1], dis, acc)
    t = _segsum_call(jnp.concatenate([p, p]), src2d, dst2d, zerosD)
    acc, p = _mid(t, W1[2], dis, acc)
    t = _segsum_call(jnp.concatenate([p, p]), src2d, dst2d, zerosD)
    acc, p = _lastprep(t, W1[3], b1r, dis, acc, W2[0])
    t = _segsum_call(jnp.concatenate([p, p]), src2d, dst2d, zerosD)
    acc, p = _mid(t, W2[1], dis, acc)
    t = _segsum_call(jnp.concatenate([p, p]), src2d, dst2d, zerosD)
    acc, p = _mid(t, W2[2], dis, acc)
    t = _segsum_call(jnp.concatenate([p, p]), src2d, dst2d, zerosD)
    out = _final(t, W2[3], b2r, dis, acc, fc_W, fcbr)
    return out


# speedup vs baseline: 1.2969x; 1.0580x over previous
import functools

import jax
import jax.numpy as jnp
from jax import lax
from jax.experimental import pallas as pl
from jax.experimental.pallas import tpu as pltpu
from jax.experimental.pallas import tpu_sc as plsc

N = 10000
E = 320000
D = 128
K = 3

NC = 2
NS = 16
CHUNK = 128
EP = 327680
EROWS = EP // CHUNK
ROWS_PER_TILE = EROWS // (NC * NS)
NPAD = 10112
ZR = NPAD // NS
OR_HI = 632
OR_LAST = N - 15 * OR_HI
DUMMY = N


HALF = ROWS_PER_TILE // 2


@functools.cache
def _sc_calls():
    mesh = plsc.VectorSubcoreMesh(core_axis_name="c", subcore_axis_name="s",
                                  num_cores=NC, num_subcores=NS)
    segsum_call = pl.kernel(
        _segsum_body,
        out_type=jax.ShapeDtypeStruct((NC, N, D), jnp.float32),
        mesh=mesh,
        scratch_types=[
            pltpu.VMEM((HALF, CHUNK), jnp.int32),
            pltpu.VMEM((HALF, CHUNK), jnp.int32),
            pltpu.VMEM((CHUNK, D), jnp.float32),
            pltpu.VMEM((CHUNK, D), jnp.float32),
            pltpu.VMEM_SHARED((NPAD, D), jnp.float32),
            pltpu.SemaphoreType.DMA,
            pltpu.SemaphoreType.DMA,
        ],
    )
    deg_call = pl.kernel(
        _deg_body,
        out_type=jax.ShapeDtypeStruct((NC, N, D), jnp.float32),
        mesh=mesh,
        scratch_types=[
            pltpu.VMEM((ROWS_PER_TILE, CHUNK), jnp.int32),
            pltpu.VMEM((CHUNK, D), jnp.float32),
            pltpu.VMEM_SHARED((NPAD, D), jnp.float32),
            pltpu.SemaphoreType.DMA,
        ],
    )
    return deg_call, segsum_call


def _segsum_body(p_hbm, src_hbm, dst_hbm, zeros_hbm, out_hbm,
                 src_v, dst_v, rows_a, rows_b, acc, sem_a, sem_b):
    c = lax.axis_index("c")
    s = lax.axis_index("s")
    w = c * NS + s
    pltpu.sync_copy(zeros_hbm.at[pl.ds(s * ZR, ZR)], acc.at[pl.ds(s * ZR, ZR)])
    plsc.subcore_barrier()

    for half in range(2):
        base = w * ROWS_PER_TILE + half * HALF
        pltpu.sync_copy(src_hbm.at[pl.ds(base, HALF)], src_v)
        pltpu.sync_copy(dst_hbm.at[pl.ds(base, HALF)], dst_v)
        pltpu.async_copy(p_hbm.at[src_v.at[0]], rows_a, sem_a)

        def body(j, carry):
            even = lax.rem(j, 2) == 0
            more = j + 1 < HALF

            @pl.when(jnp.logical_and(even, more))
            def _():
                pltpu.async_copy(p_hbm.at[src_v.at[j + 1]], rows_b, sem_b)

            @pl.when(jnp.logical_and(jnp.logical_not(even), more))
            def _():
                pltpu.async_copy(p_hbm.at[src_v.at[j + 1]], rows_a, sem_a)

            @pl.when(even)
            def _():
                pltpu.make_async_copy(p_hbm.at[src_v.at[j]], rows_a, sem_a).wait()
                pltpu.sync_copy(rows_a, acc.at[dst_v.at[j]], add=True)

            @pl.when(jnp.logical_not(even))
            def _():
                pltpu.make_async_copy(p_hbm.at[src_v.at[j]], rows_b, sem_b).wait()
                pltpu.sync_copy(rows_b, acc.at[dst_v.at[j]], add=True)

            return carry

        lax.fori_loop(0, HALF, body, 0)
    plsc.subcore_barrier()

    @pl.when(s < NS - 1)
    def _():
        pltpu.sync_copy(acc.at[pl.ds(s * OR_HI, OR_HI)],
                        out_hbm.at[c, pl.ds(s * OR_HI, OR_HI)])

    @pl.when(s == NS - 1)
    def _():
        pltpu.sync_copy(acc.at[pl.ds((NS - 1) * OR_HI, OR_LAST)],
                        out_hbm.at[c, pl.ds((NS - 1) * OR_HI, OR_LAST)])


def _deg_body(ones_hbm, dst_hbm, zeros_hbm, out_hbm, dst_v, ones_v, acc, sem):
    c = lax.axis_index("c")
    s = lax.axis_index("s")
    w = c * NS + s
    pltpu.sync_copy(zeros_hbm.at[pl.ds(s * ZR, ZR)], acc.at[pl.ds(s * ZR, ZR)])
    pltpu.sync_copy(ones_hbm, ones_v)
    pltpu.sync_copy(dst_hbm.at[pl.ds(w * ROWS_PER_TILE, ROWS_PER_TILE)], dst_v)
    plsc.subcore_barrier()

    def body(j, carry):
        pltpu.sync_copy(ones_v, acc.at[dst_v.at[j]], add=True)
        return carry

    lax.fori_loop(0, ROWS_PER_TILE, body, 0)
    plsc.subcore_barrier()

    @pl.when(s < NS - 1)
    def _():
        pltpu.sync_copy(acc.at[pl.ds(s * OR_HI, OR_HI)],
                        out_hbm.at[c, pl.ds(s * OR_HI, OR_HI)])

    @pl.when(s == NS - 1)
    def _():
        pltpu.sync_copy(acc.at[pl.ds((NS - 1) * OR_HI, OR_LAST)],
                        out_hbm.at[c, pl.ds((NS - 1) * OR_HI, OR_LAST)])


R = 400
G = N // R


def _prep0_body(deg_ref, x_ref, w_ref, acc_ref, p_ref, dis_ref):
    deg = deg_ref[0, :, 0] + deg_ref[1, :, 0]
    dis = jnp.where(deg > 0, lax.rsqrt(jnp.maximum(deg, 1.0)), 0.0)[:, None]
    x = x_ref[...]
    acc_ref[...] = jnp.dot(x, w_ref[...], preferred_element_type=jnp.float32)
    p_ref[...] = x * dis
    dis_ref[...] = dis


def _mid_body(t_ref, w_ref, dis_ref, accin_ref, accout_ref, pout_ref):
    dis = dis_ref[...]
    cur = (t_ref[0] + t_ref[1]) * dis
    accout_ref[...] = accin_ref[...] + jnp.dot(
        cur, w_ref[...], preferred_element_type=jnp.float32)
    pout_ref[...] = cur * dis


def _lastprep_body(t_ref, w_ref, b_ref, dis_ref, accin_ref, wn_ref,
                   accout_ref, pout_ref):
    dis = dis_ref[...]
    cur = (t_ref[0] + t_ref[1]) * dis
    h = accin_ref[...] + jnp.dot(cur, w_ref[...],
                                 preferred_element_type=jnp.float32) + b_ref[...]
    h = jnp.maximum(h, 0.0)
    accout_ref[...] = jnp.dot(h, wn_ref[...], preferred_element_type=jnp.float32)
    pout_ref[...] = h * dis


def _final_body(t_ref, w_ref, b_ref, dis_ref, accin_ref, fcw_ref, fcb_ref,
                out_ref):
    dis = dis_ref[...]
    cur = (t_ref[0] + t_ref[1]) * dis
    h = accin_ref[...] + jnp.dot(cur, w_ref[...],
                                 preferred_element_type=jnp.float32) + b_ref[...]
    out_ref[...] = jnp.dot(h, fcw_ref[...],
                           preferred_element_type=jnp.float32) + fcb_ref[...]


def _rows(i):
    return (i, 0)


_b_t = pl.BlockSpec((NC, R, D), lambda i: (0, i, 0))
_b_x = pl.BlockSpec((R, D), _rows)
_b_w = pl.BlockSpec((D, D), lambda i: (0, 0))
_b_dis = pl.BlockSpec((R, 1), _rows)
_b_b = pl.BlockSpec((1, D), lambda i: (0, 0))

_f32 = jnp.float32
_nd = jax.ShapeDtypeStruct((N, D), _f32)

_prep0 = pl.pallas_call(
    _prep0_body, grid=(G,),
    in_specs=[_b_t, _b_x, _b_w],
    out_specs=[_b_x, _b_x, _b_dis],
    out_shape=[_nd, _nd, jax.ShapeDtypeStruct((N, 1), _f32)],
)

_mid = pl.pallas_call(
    _mid_body, grid=(G,),
    in_specs=[_b_t, _b_w, _b_dis, _b_x],
    out_specs=[_b_x, _b_x],
    out_shape=[_nd, _nd],
)

_lastprep = pl.pallas_call(
    _lastprep_body, grid=(G,),
    in_specs=[_b_t, _b_w, _b_b, _b_dis, _b_x, _b_w],
    out_specs=[_b_x, _b_x],
    out_shape=[_nd, _nd],
)

_final = pl.pallas_call(
    _final_body, grid=(G,),
    in_specs=[_b_t, _b_w, _b_b, _b_dis, _b_x, _b_w, _b_b],
    out_specs=_b_x,
    out_shape=_nd,
)


def kernel(x, edge_index, W0, b0, W1, b1, W2, b2, fc_W, fc_b):
    src = edge_index[0].astype(jnp.int32)
    dst = edge_index[1].astype(jnp.int32)
    pad = EP - E
    src2d = jnp.concatenate([src, jnp.zeros((pad,), jnp.int32)]).reshape(EROWS, CHUNK)
    dst2d = jnp.concatenate([dst, jnp.full((pad,), DUMMY, jnp.int32)]).reshape(EROWS, CHUNK)
    zerosD = jnp.zeros((NPAD, D), _f32)
    onesCD = jnp.ones((CHUNK, D), _f32)
    _deg_call, _segsum_call = _sc_calls()

    degs = _deg_call(onesCD, dst2d, zerosD)
    b0r = b0.reshape(1, D)
    b1r = b1.reshape(1, D)
    b2r = b2.reshape(1, D)
    fcbr = fc_b.reshape(1, D)

    acc, p, dis = _prep0(degs, x, W0[0])
    t = _segsum_call(p, src2d, dst2d, zerosD)
    acc, p = _mid(t, W0[1], dis, acc)
    t = _segsum_call(p, src2d, dst2d, zerosD)
    acc, p = _mid(t, W0[2], dis, acc)
    t = _segsum_call(p, src2d, dst2d, zerosD)
    acc, p = _lastprep(t, W0[3], b0r, dis, acc, W1[0])
    t = _segsum_call(p, src2d, dst2d, zerosD)
    acc, p = _mid(t, W1[1], dis, acc)
    t = _segsum_call(p, src2d, dst2d, zerosD)
    acc, p = _mid(t, W1[2], dis, acc)
    t = _segsum_call(p, src2d, dst2d, zerosD)
    acc, p = _lastprep(t, W1[3], b1r, dis, acc, W2[0])
    t = _segsum_call(p, src2d, dst2d, zerosD)
    acc, p = _mid(t, W2[1], dis, acc)
    t = _segsum_call(p, src2d, dst2d, zerosD)
    acc, p = _mid(t, W2[2], dis, acc)
    t = _segsum_call(p, src2d, dst2d, zerosD)
    out = _final(t, W2[3], b2r, dis, acc, fc_W, fcbr)
    return out


# speedup vs baseline: 1.3082x; 1.0087x over previous
import functools

import jax
import jax.numpy as jnp
from jax import lax
from jax.experimental import pallas as pl
from jax.experimental.pallas import tpu as pltpu
from jax.experimental.pallas import tpu_sc as plsc

N = 10000
E = 320000
D = 128
K = 3

NC = 2
NS = 16
CHUNK = 128
EP = 327680
EROWS = EP // CHUNK
ROWS_PER_TILE = EROWS // (NC * NS)
NPAD = 10112
ZR = NPAD // NS
OR_HI = 632
OR_LAST = N - 15 * OR_HI
DUMMY = N


HALF = ROWS_PER_TILE // 2


@functools.cache
def _sc_calls():
    mesh = plsc.VectorSubcoreMesh(core_axis_name="c", subcore_axis_name="s",
                                  num_cores=NC, num_subcores=NS)
    segsum_call = pl.kernel(
        _segsum_body,
        out_type=jax.ShapeDtypeStruct((NC, N, D), jnp.float32),
        mesh=mesh,
        scratch_types=[
            pltpu.VMEM((HALF, CHUNK), jnp.int32),
            pltpu.VMEM((HALF, CHUNK), jnp.int32),
            pltpu.VMEM((CHUNK, D), jnp.float32),
            pltpu.VMEM((CHUNK, D), jnp.float32),
            pltpu.VMEM_SHARED((NPAD, D), jnp.float32),
            pltpu.SemaphoreType.DMA,
            pltpu.SemaphoreType.DMA,
            pltpu.SemaphoreType.DMA,
            pltpu.SemaphoreType.DMA,
        ],
    )
    deg_call = pl.kernel(
        _deg_body,
        out_type=jax.ShapeDtypeStruct((NC, N, D), jnp.float32),
        mesh=mesh,
        scratch_types=[
            pltpu.VMEM((ROWS_PER_TILE, CHUNK), jnp.int32),
            pltpu.VMEM((CHUNK, D), jnp.float32),
            pltpu.VMEM_SHARED((NPAD, D), jnp.float32),
            pltpu.SemaphoreType.DMA,
        ],
    )
    return deg_call, segsum_call


def _segsum_body(p_hbm, src_hbm, dst_hbm, zeros_hbm, out_hbm,
                 src_v, dst_v, rows_a, rows_b, acc, sem_a, sem_a2,
                 sem_b, sem_b2):
    c = lax.axis_index("c")
    s = lax.axis_index("s")
    w = c * NS + s
    pltpu.sync_copy(zeros_hbm.at[pl.ds(s * ZR, ZR)], acc.at[pl.ds(s * ZR, ZR)])
    plsc.subcore_barrier()

    H2 = CHUNK // 2

    def gather(j, rows, sem_lo, sem_hi):
        pltpu.async_copy(p_hbm.at[src_v.at[j, pl.ds(0, H2)]],
                         rows.at[pl.ds(0, H2)], sem_lo)
        pltpu.async_copy(p_hbm.at[src_v.at[j, pl.ds(H2, H2)]],
                         rows.at[pl.ds(H2, H2)], sem_hi)

    def gwait(j, rows, sem_lo, sem_hi):
        pltpu.make_async_copy(p_hbm.at[src_v.at[j, pl.ds(0, H2)]],
                              rows.at[pl.ds(0, H2)], sem_lo).wait()
        pltpu.make_async_copy(p_hbm.at[src_v.at[j, pl.ds(H2, H2)]],
                              rows.at[pl.ds(H2, H2)], sem_hi).wait()

    for half in range(2):
        base = w * ROWS_PER_TILE + half * HALF
        pltpu.sync_copy(src_hbm.at[pl.ds(base, HALF)], src_v)
        pltpu.sync_copy(dst_hbm.at[pl.ds(base, HALF)], dst_v)
        gather(0, rows_a, sem_a, sem_a2)

        def body(j, carry):
            even = lax.rem(j, 2) == 0
            more = j + 1 < HALF

            @pl.when(jnp.logical_and(even, more))
            def _():
                gather(j + 1, rows_b, sem_b, sem_b2)

            @pl.when(jnp.logical_and(jnp.logical_not(even), more))
            def _():
                gather(j + 1, rows_a, sem_a, sem_a2)

            @pl.when(even)
            def _():
                gwait(j, rows_a, sem_a, sem_a2)
                pltpu.sync_copy(rows_a, acc.at[dst_v.at[j]], add=True)

            @pl.when(jnp.logical_not(even))
            def _():
                gwait(j, rows_b, sem_b, sem_b2)
                pltpu.sync_copy(rows_b, acc.at[dst_v.at[j]], add=True)

            return carry

        lax.fori_loop(0, HALF, body, 0)
    plsc.subcore_barrier()

    @pl.when(s < NS - 1)
    def _():
        pltpu.sync_copy(acc.at[pl.ds(s * OR_HI, OR_HI)],
                        out_hbm.at[c, pl.ds(s * OR_HI, OR_HI)])

    @pl.when(s == NS - 1)
    def _():
        pltpu.sync_copy(acc.at[pl.ds((NS - 1) * OR_HI, OR_LAST)],
                        out_hbm.at[c, pl.ds((NS - 1) * OR_HI, OR_LAST)])


def _deg_body(ones_hbm, dst_hbm, zeros_hbm, out_hbm, dst_v, ones_v, acc, sem):
    c = lax.axis_index("c")
    s = lax.axis_index("s")
    w = c * NS + s
    pltpu.sync_copy(zeros_hbm.at[pl.ds(s * ZR, ZR)], acc.at[pl.ds(s * ZR, ZR)])
    pltpu.sync_copy(ones_hbm, ones_v)
    pltpu.sync_copy(dst_hbm.at[pl.ds(w * ROWS_PER_TILE, ROWS_PER_TILE)], dst_v)
    plsc.subcore_barrier()

    def body(j, carry):
        pltpu.sync_copy(ones_v, acc.at[dst_v.at[j]], add=True)
        return carry

    lax.fori_loop(0, ROWS_PER_TILE, body, 0)
    plsc.subcore_barrier()

    @pl.when(s < NS - 1)
    def _():
        pltpu.sync_copy(acc.at[pl.ds(s * OR_HI, OR_HI)],
                        out_hbm.at[c, pl.ds(s * OR_HI, OR_HI)])

    @pl.when(s == NS - 1)
    def _():
        pltpu.sync_copy(acc.at[pl.ds((NS - 1) * OR_HI, OR_LAST)],
                        out_hbm.at[c, pl.ds((NS - 1) * OR_HI, OR_LAST)])


R = 400
G = N // R


def _prep0_body(deg_ref, x_ref, w_ref, acc_ref, p_ref, dis_ref):
    deg = deg_ref[0, :, 0] + deg_ref[1, :, 0]
    dis = jnp.where(deg > 0, lax.rsqrt(jnp.maximum(deg, 1.0)), 0.0)[:, None]
    x = x_ref[...]
    acc_ref[...] = jnp.dot(x, w_ref[...], preferred_element_type=jnp.float32)
    p_ref[...] = x * dis
    dis_ref[...] = dis


def _mid_body(t_ref, w_ref, dis_ref, accin_ref, accout_ref, pout_ref):
    dis = dis_ref[...]
    cur = (t_ref[0] + t_ref[1]) * dis
    accout_ref[...] = accin_ref[...] + jnp.dot(
        cur, w_ref[...], preferred_element_type=jnp.float32)
    pout_ref[...] = cur * dis


def _lastprep_body(t_ref, w_ref, b_ref, dis_ref, accin_ref, wn_ref,
                   accout_ref, pout_ref):
    dis = dis_ref[...]
    cur = (t_ref[0] + t_ref[1]) * dis
    h = accin_ref[...] + jnp.dot(cur, w_ref[...],
                                 preferred_element_type=jnp.float32) + b_ref[...]
    h = jnp.maximum(h, 0.0)
    accout_ref[...] = jnp.dot(h, wn_ref[...], preferred_element_type=jnp.float32)
    pout_ref[...] = h * dis


def _final_body(t_ref, w_ref, b_ref, dis_ref, accin_ref, fcw_ref, fcb_ref,
                out_ref):
    dis = dis_ref[...]
    cur = (t_ref[0] + t_ref[1]) * dis
    h = accin_ref[...] + jnp.dot(cur, w_ref[...],
                                 preferred_element_type=jnp.float32) + b_ref[...]
    out_ref[...] = jnp.dot(h, fcw_ref[...],
                           preferred_element_type=jnp.float32) + fcb_ref[...]


def _rows(i):
    return (i, 0)


_b_t = pl.BlockSpec((NC, R, D), lambda i: (0, i, 0))
_b_x = pl.BlockSpec((R, D), _rows)
_b_w = pl.BlockSpec((D, D), lambda i: (0, 0))
_b_dis = pl.BlockSpec((R, 1), _rows)
_b_b = pl.BlockSpec((1, D), lambda i: (0, 0))

_f32 = jnp.float32
_nd = jax.ShapeDtypeStruct((N, D), _f32)

_prep0 = pl.pallas_call(
    _prep0_body, grid=(G,),
    in_specs=[_b_t, _b_x, _b_w],
    out_specs=[_b_x, _b_x, _b_dis],
    out_shape=[_nd, _nd, jax.ShapeDtypeStruct((N, 1), _f32)],
)

_mid = pl.pallas_call(
    _mid_body, grid=(G,),
    in_specs=[_b_t, _b_w, _b_dis, _b_x],
    out_specs=[_b_x, _b_x],
    out_shape=[_nd, _nd],
)

_lastprep = pl.pallas_call(
    _lastprep_body, grid=(G,),
    in_specs=[_b_t, _b_w, _b_b, _b_dis, _b_x, _b_w],
    out_specs=[_b_x, _b_x],
    out_shape=[_nd, _nd],
)

_final = pl.pallas_call(
    _final_body, grid=(G,),
    in_specs=[_b_t, _b_w, _b_b, _b_dis, _b_x, _b_w, _b_b],
    out_specs=_b_x,
    out_shape=_nd,
)


def kernel(x, edge_index, W0, b0, W1, b1, W2, b2, fc_W, fc_b):
    src = edge_index[0].astype(jnp.int32)
    dst = edge_index[1].astype(jnp.int32)
    pad = EP - E
    src2d = jnp.concatenate([src, jnp.zeros((pad,), jnp.int32)]).reshape(EROWS, CHUNK)
    dst2d = jnp.concatenate([dst, jnp.full((pad,), DUMMY, jnp.int32)]).reshape(EROWS, CHUNK)
    zerosD = jnp.zeros((NPAD, D), _f32)
    onesCD = jnp.ones((CHUNK, D), _f32)
    _deg_call, _segsum_call = _sc_calls()

    degs = _deg_call(onesCD, dst2d, zerosD)
    b0r = b0.reshape(1, D)
    b1r = b1.reshape(1, D)
    b2r = b2.reshape(1, D)
    fcbr = fc_b.reshape(1, D)

    acc, p, dis = _prep0(degs, x, W0[0])
    t = _segsum_call(p, src2d, dst2d, zerosD)
    acc, p = _mid(t, W0[1], dis, acc)
    t = _segsum_call(p, src2d, dst2d, zerosD)
    acc, p = _mid(t, W0[2], dis, acc)
    t = _segsum_call(p, src2d, dst2d, zerosD)
    acc, p = _lastprep(t, W0[3], b0r, dis, acc, W1[0])
    t = _segsum_call(p, src2d, dst2d, zerosD)
    acc, p = _mid(t, W1[1], dis, acc)
    t = _segsum_call(p, src2d, dst2d, zerosD)
    acc, p = _mid(t, W1[2], dis, acc)
    t = _segsum_call(p, src2d, dst2d, zerosD)
    acc, p = _lastprep(t, W1[3], b1r, dis, acc, W2[0])
    t = _segsum_call(p, src2d, dst2d, zerosD)
    acc, p = _mid(t, W2[1], dis, acc)
    t = _segsum_call(p, src2d, dst2d, zerosD)
    acc, p = _mid(t, W2[2], dis, acc)
    t = _segsum_call(p, src2d, dst2d, zerosD)
    out = _final(t, W2[3], b2r, dis, acc, fc_W, fcbr)
    return out
